# baseline (speedup 1.0000x reference)
import functools

import jax
import jax.numpy as jnp
from jax import lax
from jax.experimental import pallas as pl
from jax.experimental.pallas import tpu as pltpu
from jax.experimental.pallas import tpu_sc as plsc

_NC = 2
_NS = 16
_NW = _NC * _NS
_SC_PARAMS = pltpu.CompilerParams(use_tc_tiling_on_sc=False, needs_layout_passes=False)


def _mesh():
    return plsc.VectorSubcoreMesh(core_axis_name="c", subcore_axis_name="s")


def _rcp(x):
    i = plsc.bitcast(x, jnp.int32)
    y = plsc.bitcast(jnp.full(i.shape, 0x7EF311C3, jnp.int32) - i, jnp.float32)
    y = y * (2.0 - x * y)
    y = y * (2.0 - x * y)
    return y


def _rcp1(x):
    i = plsc.bitcast(x, jnp.int32)
    y = plsc.bitcast(jnp.full(i.shape, 0x7EF311C3, jnp.int32) - i, jnp.float32)
    return y * (2.0 - x * y)


def _wid():
    return lax.axis_index("s") * _NC + lax.axis_index("c")


def _sc_gather_rows(table, idx, rows_per_worker, dim):
    n = idx.shape[0]

    @functools.partial(
        pl.kernel,
        out_type=jax.ShapeDtypeStruct((n, dim), jnp.float32),
        mesh=_mesh(),
        scratch_types=[
            pltpu.VMEM((rows_per_worker,), jnp.int32),
            pltpu.VMEM((rows_per_worker, dim), jnp.float32),
            pltpu.SemaphoreType.DMA,
        ],
        compiler_params=_SC_PARAMS,
    )
    def gk(table_hbm, idx_hbm, out_hbm, idx_v, rows_v, sem):
        base = _wid() * rows_per_worker
        pltpu.sync_copy(idx_hbm.at[pl.ds(base, rows_per_worker)], idx_v)
        pltpu.async_copy(table_hbm.at[idx_v], rows_v, sem).wait()
        pltpu.sync_copy(rows_v, out_hbm.at[pl.ds(base, rows_per_worker)])

    return gk(table, idx)


def _sc_gru_mean(g_rel_tab, g_ent, rel_ids, nnum, bhn2):
    npairs, maxn = rel_ids.shape
    ppw = npairs // _NW

    @functools.partial(
        pl.kernel,
        out_type=jax.ShapeDtypeStruct((npairs, 64), jnp.float32),
        mesh=_mesh(),
        scratch_types=[
            pltpu.VMEM((ppw, maxn), jnp.int32),
            pltpu.VMEM((ppw, 192), jnp.float32),
            pltpu.VMEM((ppw,), jnp.int32),
            pltpu.VMEM((maxn, 192), jnp.float32),
            pltpu.VMEM((maxn, 192), jnp.float32),
            pltpu.VMEM((ppw, 64), jnp.float32),
            pltpu.VMEM((64,), jnp.float32),
            pltpu.SemaphoreType.DMA,
            pltpu.SemaphoreType.DMA,
        ],
        compiler_params=_SC_PARAMS,
    )
    def gru_k(greltab_hbm, gent_hbm, relids_hbm, nnum_hbm, bhn2_hbm, out_hbm,
              ids_v, gent_v, nn_v, grel0_v, grel1_v, upd_v, bhn_v, sem0, sem1):
        base = _wid() * ppw
        pltpu.sync_copy(relids_hbm.at[pl.ds(base, ppw)], ids_v)
        pltpu.sync_copy(gent_hbm.at[pl.ds(base, ppw)], gent_v)
        pltpu.sync_copy(nnum_hbm.at[pl.ds(base, ppw)], nn_v)
        pltpu.sync_copy(bhn2_hbm, bhn_v)
        bh = [bhn_v[pl.ds(j * 16, 16)] for j in range(4)]
        bufs = (grel0_v, grel1_v)
        sems = (sem0, sem1)

        pltpu.async_copy(greltab_hbm.at[ids_v.at[0]], grel0_v, sem0)

        def p2_body(p2, _):
            for half in range(2):
                p = p2 * 2 + half
                grel_v = bufs[half]
                pltpu.make_async_copy(
                    greltab_hbm.at[pl.ds(0, maxn)], grel_v, sems[half]).wait()

                @pl.when(p + 1 < ppw)
                def _():
                    pltpu.async_copy(greltab_hbm.at[ids_v.at[p + 1]],
                                     bufs[1 - half], sems[1 - half])

                nn16 = plsc.load_gather(nn_v, [jnp.full((16,), p, jnp.int32)])
                nn = nn16[0]
                ge = [gent_v[p, pl.ds(j * 16, 16)] for j in range(12)]

                def m_body(m, accs):
                    out = []
                    for j in range(4):
                        pr = grel_v[m, pl.ds(j * 16, 16)] + ge[j]
                        pz = grel_v[m, pl.ds(64 + j * 16, 16)] + ge[4 + j]
                        pn = grel_v[m, pl.ds(128 + j * 16, 16)] + ge[8 + j]
                        r = _rcp1(1.0 + jnp.exp(pr))
                        e2 = jnp.exp(jnp.minimum(pn + r * bh[j], 60.0))
                        t = (e2 - 1.0) * _rcp((e2 + 1.0) * (1.0 + jnp.exp(pz)))
                        out.append(accs[j] + t)
                    return tuple(out)

                z16 = jnp.zeros((16,), jnp.float32)
                accs = lax.fori_loop(0, nn, m_body, (z16, z16, z16, z16))
                inv16 = _rcp(jnp.maximum(nn16.astype(jnp.float32), 1.0))
                for j in range(4):
                    upd_v[p, pl.ds(j * 16, 16)] = accs[j] * inv16
            return 0

        lax.fori_loop(0, ppw // 2, p2_body, 0)
        pltpu.sync_copy(upd_v, out_hbm.at[pl.ds(base, ppw)])

    return gru_k(g_rel_tab, g_ent, rel_ids, nnum, bhn2)


def _sc_score(ent_proj, rel_proj, upd33, slot_ids, cent_ids, crel_ids,
              cur_state):
    B, NE = cent_ids.shape
    bpw = B // _NW
    CH = 256
    nch = NE // CH

    @functools.partial(
        pl.kernel,
        out_type=jax.ShapeDtypeStruct((B, NE), jnp.float32),
        mesh=_mesh(),
        scratch_types=[
            pltpu.VMEM((33, 64), jnp.float32),
            pltpu.VMEM((32, 64), jnp.float32),
            pltpu.VMEM((NE,), jnp.int32),
            pltpu.VMEM((NE,), jnp.int32),
            pltpu.VMEM((NE,), jnp.int32),
            pltpu.VMEM((CH, 64), jnp.float32),
            pltpu.VMEM((CH, 64), jnp.float32),
            pltpu.VMEM((CH, 64), jnp.float32),
            pltpu.VMEM((CH, 64), jnp.float32),
            pltpu.VMEM((NE,), jnp.float32),
            pltpu.SemaphoreType.DMA,
            pltpu.SemaphoreType.DMA,
            pltpu.SemaphoreType.DMA,
            pltpu.SemaphoreType.DMA,
        ],
        compiler_params=_SC_PARAMS,
    )
    def score_k(entp_hbm, relp_hbm, upd_hbm, slot_hbm, cent_hbm, crel_hbm,
                cur_hbm, out_hbm, upd_v, cur_v,
                sid_v, eid_v, rid_v, erow0_v, erow1_v, rrow0_v, rrow1_v,
                sc_v, sem_e0, sem_e1, sem_r0, sem_r1):
        wid = _wid()
        ebufs, rbufs = (erow0_v, erow1_v), (rrow0_v, rrow1_v)
        esems, rsems = (sem_e0, sem_e1), (sem_r0, sem_r1)

        def issue(c, bi):
            pltpu.async_copy(
                entp_hbm.at[eid_v.at[pl.ds(c * CH, CH)]], ebufs[bi], esems[bi])
            pltpu.async_copy(
                relp_hbm.at[rid_v.at[pl.ds(c * CH, CH)]], rbufs[bi], rsems[bi])

        def batch_body(k, _):
            b = wid * bpw + k
            pltpu.sync_copy(upd_hbm.at[pl.ds(b * 33, 33)], upd_v)
            pltpu.sync_copy(cur_hbm.at[pl.ds(b * 32, 32)], cur_v)
            pltpu.sync_copy(slot_hbm.at[b], sid_v)
            pltpu.sync_copy(cent_hbm.at[b], eid_v)
            pltpu.sync_copy(crel_hbm.at[b], rid_v)

            lane = lax.iota(jnp.int32, 16)
            issue(0, 0)
            for c in range(nch):
                bi = c & 1
                erow_v, rrow_v = ebufs[bi], rbufs[bi]
                pltpu.make_async_copy(
                    entp_hbm.at[pl.ds(0, CH)], erow_v, esems[bi]).wait()
                pltpu.make_async_copy(
                    relp_hbm.at[pl.ds(0, CH)], rrow_v, rsems[bi]).wait()
                if c + 1 < nch:
                    issue(c + 1, 1 - bi)

                def group_body(gg, _, c=c, erow_v=erow_v, rrow_v=rrow_v):
                    e0 = c * CH + gg * 16
                    slot16 = sid_v[pl.ds(e0, 16)]
                    loc16 = lane + gg * 16
                    r_row = lax.shift_right_logical(c * CH + gg * 16, 6)
                    rr16 = jnp.full((16,), r_row, jnp.int32)

                    def d_body(i, acc):
                        for dd in range(8):
                            cols = lax.bitwise_and(lane + (i * 8 + dd), 63)
                            ev = plsc.load_gather(erow_v, [loc16, cols])
                            rv = plsc.load_gather(rrow_v, [loc16, cols])
                            uv = plsc.load_gather(upd_v, [slot16, cols])
                            cd = plsc.load_gather(cur_v, [rr16, cols])
                            v = ev + rv + uv
                            cv = jnp.where(v >= 0.0, v, 0.01 * v)
                            acc = acc + cv * cd
                        return acc

                    acc = lax.fori_loop(0, 8, d_body, jnp.zeros((16,), jnp.float32))
                    sc_v[pl.ds(e0, 16)] = acc
                    return 0

                lax.fori_loop(0, CH // 16, group_body, 0)
            pltpu.sync_copy(sc_v, out_hbm.at[b])
            return 0

        lax.fori_loop(0, bpw, batch_body, 0)

    return score_k(ent_proj, rel_proj, upd33, slot_ids, cent_ids, crel_ids,
                   cur_state)


def _tc1_body(rows_aim_ref, rows_cur_ref, rel_ref, q_ref,
              wents_ref, wrels_ref, biass_ref, wn2_ref, wn3_ref, bn_ref,
              wc3_ref, bc_ref,
              gent_ref, greltab_ref, relproj_ref, curpart_ref):
    gent_ref[...] = rows_aim_ref[...] @ wents_ref[...] + biass_ref[...]
    greltab_ref[...] = rel_ref[...] @ wrels_ref[...]
    relproj_ref[...] = rel_ref[...] @ wc3_ref[...] + bc_ref[...]
    qp = q_ref[...] @ wn2_ref[...] + bn_ref[...]
    B = qp.shape[0]
    cp = rows_cur_ref[...] @ wn3_ref[...]
    curpart_ref[...] = cp.reshape(B, 32, 64) + qp[:, None, :]


def _tc2_body(et_ref, wc2_ref, out_ref):
    out_ref[...] = et_ref[...] @ wc2_ref[...]


def _tc3_body(upd_ref, curn_ref, aim_ref, curpart_ref, wc1_ref, wn1_ref,
              nid_ref, upd33_ref, cur_ref, slot_ref):
    B = aim_ref.shape[0]
    upd = upd_ref[...].reshape(B * 32, 64)
    upc = (upd @ wc1_ref[...]).reshape(B, 32, 64)
    upn = (upd @ wn1_ref[...]).reshape(B, 32, 64)
    upd33_ref[...] = jnp.concatenate(
        [upc, jnp.zeros((B, 1, 64), jnp.float32)], axis=1)
    eqn = nid_ref[...][:, :, None] == aim_ref[...][:, None, :]
    tp1 = lax.broadcasted_iota(jnp.int32, eqn.shape, 2) + 1
    mx = jnp.max(jnp.where(eqn, tp1, 0), axis=-1)
    slot_ref[...] = jnp.where(mx == 0, 32, mx - 1)
    eq = (curn_ref[...][:, :, None] == aim_ref[...][:, None, :]).astype(jnp.float32)
    tt = lax.broadcasted_iota(jnp.int32, (32, 32), 0)
    uu = (tt > lax.broadcasted_iota(jnp.int32, (32, 32), 1)).astype(jnp.float32)
    sfx = jax.lax.dot_general(eq, uu, (((2,), (0,)), ((), ())),
                              preferred_element_type=jnp.float32)
    onehot_last = eq * (sfx == 0.0).astype(jnp.float32)
    rep = jax.lax.dot_general(onehot_last, upn, (((2,), (1,)), ((0,), (0,))),
                              preferred_element_type=jnp.float32)
    pre = curpart_ref[...] + rep
    cur_ref[...] = jnp.where(pre >= 0.0, pre, 0.01 * pre) * 0.125


def _full_spec(shape):
    return pl.BlockSpec(shape, lambda: tuple(0 for _ in shape))


def kernel(start_entities, query_representations, aim_nodes, aim_entities,
           aim_nums, neighbors, neighbors_num, current_nodes, current_entities,
           current_masks, candidate_nodes, candidate_entities,
           candidate_relations, candidate_masks, entity_table, relation_table,
           W_ih, W_hh, b_ih, b_hh, W_next, b_next, W_cand, b_cand):
    B, TOPK = aim_nodes.shape
    ROLL = current_nodes.shape[1]
    MAXN = neighbors.shape[2]
    E = entity_table.shape[1]
    H = W_hh.shape[1]
    V = entity_table.shape[0]
    NREL = relation_table.shape[0]
    MAX_NODES = 2048
    f32 = jnp.float32

    i32 = jnp.int32
    idx_all = jnp.concatenate(
        [aim_entities.reshape(-1), current_entities.reshape(-1)]).astype(i32)
    rel_ids = neighbors[..., 1].reshape(B * TOPK, MAXN).astype(i32)
    nnum = neighbors_num.reshape(-1).astype(i32)
    cent_ids = candidate_entities.reshape(B, ROLL * MAXN).astype(i32)
    crel_ids = candidate_relations.reshape(B, ROLL * MAXN).astype(i32)
    cnode_ids = candidate_nodes.reshape(B, ROLL * MAXN).astype(i32)
    aim_i = aim_nodes.astype(i32)
    curn_i = current_nodes.astype(i32)

    scale = jnp.concatenate([-jnp.ones((H,), f32), jnp.ones((H,), f32),
                             2.0 * jnp.ones((H,), f32)])
    bias = b_ih + jnp.concatenate([b_hh[:2 * H], jnp.zeros((H,), f32)])
    W_ent_s = (W_ih[:, :E] * scale[:, None]).T
    W_rel_s = (W_ih[:, E:] * scale[:, None]).T
    bias_s = (bias * scale)[None, :]
    bhn2 = 2.0 * b_hh[2 * H:]
    Wn1T, Wn2T, Wn3T = (W_next[:, :H].T, W_next[:, H:H + 64].T,
                        W_next[:, H + 64:].T)
    Wc1T, Wc2T, Wc3T = (W_cand[:, :H].T, W_cand[:, H:H + E].T,
                        W_cand[:, H + E:].T)

    rows = _sc_gather_rows(entity_table, idx_all, idx_all.shape[0] // _NW, E)
    rows_aim = rows[:B * TOPK]
    rows_cur = rows[B * TOPK:]

    g_ent, g_rel_tab, rel_proj, cur_part = pl.pallas_call(
        _tc1_body,
        in_specs=[_full_spec((B * TOPK, E)), _full_spec((B * ROLL, E)),
                  _full_spec((NREL, E)), _full_spec((B, 64)),
                  _full_spec((E, 192)), _full_spec((E, 192)),
                  _full_spec((1, 192)), _full_spec((64, 64)),
                  _full_spec((64, 64)), _full_spec((1, 64)),
                  _full_spec((64, 64)), _full_spec((1, 64))],
        out_specs=[_full_spec((B * TOPK, 192)), _full_spec((NREL, 192)),
                   _full_spec((NREL, 64)), _full_spec((B, ROLL, 64))],
        out_shape=[jax.ShapeDtypeStruct((B * TOPK, 192), f32),
                   jax.ShapeDtypeStruct((NREL, 192), f32),
                   jax.ShapeDtypeStruct((NREL, 64), f32),
                   jax.ShapeDtypeStruct((B, ROLL, 64), f32)],
    )(rows_aim, rows_cur, relation_table, query_representations,
      W_ent_s, W_rel_s, bias_s, Wn2T, Wn3T, b_next[None, :],
      Wc3T, b_cand[None, :])

    RB = 800
    ent_proj = pl.pallas_call(
        _tc2_body,
        grid=(V // RB,),
        in_specs=[pl.BlockSpec((RB, E), lambda i: (i, 0)),
                  pl.BlockSpec((E, 64), lambda i: (0, 0))],
        out_specs=pl.BlockSpec((RB, 64), lambda i: (i, 0)),
        out_shape=jax.ShapeDtypeStruct((V, 64), f32),
    )(entity_table, Wc2T)

    upd = _sc_gru_mean(g_rel_tab, g_ent, rel_ids, nnum, bhn2)

    SB = 8
    upd33, cur_state, slot_ids = pl.pallas_call(
        _tc3_body,
        grid=(B // SB,),
        in_specs=[pl.BlockSpec((SB, TOPK, 64), lambda i: (i, 0, 0)),
                  pl.BlockSpec((SB, ROLL), lambda i: (i, 0)),
                  pl.BlockSpec((SB, TOPK), lambda i: (i, 0)),
                  pl.BlockSpec((SB, ROLL, 64), lambda i: (i, 0, 0)),
                  pl.BlockSpec((64, 64), lambda i: (0, 0)),
                  pl.BlockSpec((64, 64), lambda i: (0, 0)),
                  pl.BlockSpec((SB, ROLL * MAXN), lambda i: (i, 0))],
        out_specs=[pl.BlockSpec((SB, TOPK + 1, 64), lambda i: (i, 0, 0)),
                   pl.BlockSpec((SB, ROLL, 64), lambda i: (i, 0, 0)),
                   pl.BlockSpec((SB, ROLL * MAXN), lambda i: (i, 0))],
        out_shape=[jax.ShapeDtypeStruct((B, TOPK + 1, 64), f32),
                   jax.ShapeDtypeStruct((B, ROLL, 64), f32),
                   jax.ShapeDtypeStruct((B, ROLL * MAXN), i32)],
    )(upd.reshape(B, TOPK, 64), curn_i, aim_i, cur_part, Wc1T, Wn1T,
      cnode_ids)

    scores = _sc_score(ent_proj, rel_proj, upd33.reshape(B * (TOPK + 1), 64),
                       slot_ids, cent_ids, crel_ids,
                       cur_state.reshape(B * ROLL, 64))

    scores = scores.reshape(B, ROLL, MAXN)
    scores = jnp.where(~candidate_masks, -100000.0, scores)
    return scores


# speedup vs baseline: 1.0047x; 1.0047x over previous
import functools

import jax
import jax.numpy as jnp
from jax import lax
from jax.experimental import pallas as pl
from jax.experimental.pallas import tpu as pltpu
from jax.experimental.pallas import tpu_sc as plsc

_NC = 2
_NS = 16
_NW = _NC * _NS
_SC_PARAMS = pltpu.CompilerParams(use_tc_tiling_on_sc=False, needs_layout_passes=False)


def _mesh():
    return plsc.VectorSubcoreMesh(core_axis_name="c", subcore_axis_name="s")


def _rcp(x):
    i = plsc.bitcast(x, jnp.int32)
    y = plsc.bitcast(jnp.full(i.shape, 0x7EF311C3, jnp.int32) - i, jnp.float32)
    y = y * (2.0 - x * y)
    y = y * (2.0 - x * y)
    return y


def _rcp1(x):
    i = plsc.bitcast(x, jnp.int32)
    y = plsc.bitcast(jnp.full(i.shape, 0x7EF311C3, jnp.int32) - i, jnp.float32)
    return y * (2.0 - x * y)


def _wid():
    return lax.axis_index("s") * _NC + lax.axis_index("c")


def _sc_gather_rows(table, idx, rows_per_worker, dim):
    n = idx.shape[0]

    @functools.partial(
        pl.kernel,
        out_type=jax.ShapeDtypeStruct((n, dim), jnp.float32),
        mesh=_mesh(),
        scratch_types=[
            pltpu.VMEM((rows_per_worker,), jnp.int32),
            pltpu.VMEM((rows_per_worker, dim), jnp.float32),
            pltpu.SemaphoreType.DMA,
        ],
        compiler_params=_SC_PARAMS,
    )
    def gk(table_hbm, idx_hbm, out_hbm, idx_v, rows_v, sem):
        base = _wid() * rows_per_worker
        pltpu.sync_copy(idx_hbm.at[pl.ds(base, rows_per_worker)], idx_v)
        pltpu.async_copy(table_hbm.at[idx_v], rows_v, sem).wait()
        pltpu.sync_copy(rows_v, out_hbm.at[pl.ds(base, rows_per_worker)])

    return gk(table, idx)


def _sc_gru_mean(g_rel_tab, g_ent, rel_ids, nnum, bhn2):
    npairs, maxn = rel_ids.shape
    ppw = npairs // _NW

    @functools.partial(
        pl.kernel,
        out_type=jax.ShapeDtypeStruct((npairs, 64), jnp.float32),
        mesh=_mesh(),
        scratch_types=[
            pltpu.VMEM((ppw, maxn), jnp.int32),
            pltpu.VMEM((ppw, 192), jnp.float32),
            pltpu.VMEM((ppw,), jnp.int32),
            pltpu.VMEM((maxn, 192), jnp.float32),
            pltpu.VMEM((maxn, 192), jnp.float32),
            pltpu.VMEM((ppw, 64), jnp.float32),
            pltpu.VMEM((64,), jnp.float32),
            pltpu.SemaphoreType.DMA,
            pltpu.SemaphoreType.DMA,
        ],
        compiler_params=_SC_PARAMS,
    )
    def gru_k(greltab_hbm, gent_hbm, relids_hbm, nnum_hbm, bhn2_hbm, out_hbm,
              ids_v, gent_v, nn_v, grel0_v, grel1_v, upd_v, bhn_v, sem0, sem1):
        base = _wid() * ppw
        pltpu.sync_copy(relids_hbm.at[pl.ds(base, ppw)], ids_v)
        pltpu.sync_copy(gent_hbm.at[pl.ds(base, ppw)], gent_v)
        pltpu.sync_copy(nnum_hbm.at[pl.ds(base, ppw)], nn_v)
        pltpu.sync_copy(bhn2_hbm, bhn_v)
        bh = [bhn_v[pl.ds(j * 16, 16)] for j in range(4)]
        bufs = (grel0_v, grel1_v)
        sems = (sem0, sem1)

        pltpu.async_copy(greltab_hbm.at[ids_v.at[0]], grel0_v, sem0)

        def p2_body(p2, _):
            for half in range(2):
                p = p2 * 2 + half
                grel_v = bufs[half]
                pltpu.make_async_copy(
                    greltab_hbm.at[pl.ds(0, maxn)], grel_v, sems[half]).wait()

                @pl.when(p + 1 < ppw)
                def _():
                    pltpu.async_copy(greltab_hbm.at[ids_v.at[p + 1]],
                                     bufs[1 - half], sems[1 - half])

                nn16 = plsc.load_gather(nn_v, [jnp.full((16,), p, jnp.int32)])
                nn = nn16[0]
                ge = [gent_v[p, pl.ds(j * 16, 16)] for j in range(12)]

                def gate(m, j):
                    pr = grel_v[m, pl.ds(j * 16, 16)] + ge[j]
                    pz = grel_v[m, pl.ds(64 + j * 16, 16)] + ge[4 + j]
                    pn = grel_v[m, pl.ds(128 + j * 16, 16)] + ge[8 + j]
                    r = _rcp1(1.0 + jnp.exp(pr))
                    e2 = jnp.exp(jnp.minimum(pn + r * bh[j], 60.0))
                    return (e2 - 1.0) * _rcp1((e2 + 1.0) * (1.0 + jnp.exp(pz)))

                def m2_body(i, accs):
                    m = i * 2
                    return tuple(accs[j] + gate(m, j) + gate(m + 1, j)
                                 for j in range(4))

                z16 = jnp.zeros((16,), jnp.float32)
                accs = lax.fori_loop(
                    0, lax.shift_right_logical(nn, 1), m2_body,
                    (z16, z16, z16, z16))
                modd = jnp.maximum(nn - 1, 0)
                oddf = (lax.bitwise_and(nn, 1)).astype(jnp.float32)
                accs = tuple(accs[j] + oddf * gate(modd, j) for j in range(4))
                inv16 = _rcp(jnp.maximum(nn16.astype(jnp.float32), 1.0))
                for j in range(4):
                    upd_v[p, pl.ds(j * 16, 16)] = accs[j] * inv16
            return 0

        lax.fori_loop(0, ppw // 2, p2_body, 0)
        pltpu.sync_copy(upd_v, out_hbm.at[pl.ds(base, ppw)])

    return gru_k(g_rel_tab, g_ent, rel_ids, nnum, bhn2)


def _sc_score(ent_proj, rel_proj, upd33, slot_ids, cent_ids, crel_ids,
              cur_state):
    B, NE = cent_ids.shape
    bpw = B // _NW
    CH = 256
    nch = NE // CH

    @functools.partial(
        pl.kernel,
        out_type=jax.ShapeDtypeStruct((B, NE), jnp.float32),
        mesh=_mesh(),
        scratch_types=[
            pltpu.VMEM((33, 64), jnp.float32),
            pltpu.VMEM((32, 64), jnp.float32),
            pltpu.VMEM((NE,), jnp.int32),
            pltpu.VMEM((NE,), jnp.int32),
            pltpu.VMEM((NE,), jnp.int32),
            pltpu.VMEM((CH, 64), jnp.float32),
            pltpu.VMEM((CH, 64), jnp.float32),
            pltpu.VMEM((CH, 64), jnp.float32),
            pltpu.VMEM((CH, 64), jnp.float32),
            pltpu.VMEM((NE,), jnp.float32),
            pltpu.SemaphoreType.DMA,
            pltpu.SemaphoreType.DMA,
            pltpu.SemaphoreType.DMA,
            pltpu.SemaphoreType.DMA,
        ],
        compiler_params=_SC_PARAMS,
    )
    def score_k(entp_hbm, relp_hbm, upd_hbm, slot_hbm, cent_hbm, crel_hbm,
                cur_hbm, out_hbm, upd_v, cur_v,
                sid_v, eid_v, rid_v, erow0_v, erow1_v, rrow0_v, rrow1_v,
                sc_v, sem_e0, sem_e1, sem_r0, sem_r1):
        wid = _wid()
        ebufs, rbufs = (erow0_v, erow1_v), (rrow0_v, rrow1_v)
        esems, rsems = (sem_e0, sem_e1), (sem_r0, sem_r1)

        def issue(c, bi):
            pltpu.async_copy(
                entp_hbm.at[eid_v.at[pl.ds(c * CH, CH)]], ebufs[bi], esems[bi])
            pltpu.async_copy(
                relp_hbm.at[rid_v.at[pl.ds(c * CH, CH)]], rbufs[bi], rsems[bi])

        def batch_body(k, _):
            b = wid * bpw + k
            pltpu.sync_copy(upd_hbm.at[pl.ds(b * 33, 33)], upd_v)
            pltpu.sync_copy(cur_hbm.at[pl.ds(b * 32, 32)], cur_v)
            pltpu.sync_copy(slot_hbm.at[b], sid_v)
            pltpu.sync_copy(cent_hbm.at[b], eid_v)
            pltpu.sync_copy(crel_hbm.at[b], rid_v)

            lane = lax.iota(jnp.int32, 16)
            issue(0, 0)
            for c in range(nch):
                bi = c & 1
                erow_v, rrow_v = ebufs[bi], rbufs[bi]
                pltpu.make_async_copy(
                    entp_hbm.at[pl.ds(0, CH)], erow_v, esems[bi]).wait()
                pltpu.make_async_copy(
                    relp_hbm.at[pl.ds(0, CH)], rrow_v, rsems[bi]).wait()
                if c + 1 < nch:
                    issue(c + 1, 1 - bi)

                def group_body(gg, _, c=c, erow_v=erow_v, rrow_v=rrow_v):
                    e0 = c * CH + gg * 16
                    slot16 = sid_v[pl.ds(e0, 16)]
                    loc16 = lane + gg * 16
                    r_row = lax.shift_right_logical(c * CH + gg * 16, 6)
                    rr16 = jnp.full((16,), r_row, jnp.int32)

                    def d_body(i, acc):
                        for dd in range(8):
                            cols = lax.bitwise_and(lane + (i * 8 + dd), 63)
                            ev = plsc.load_gather(erow_v, [loc16, cols])
                            rv = plsc.load_gather(rrow_v, [loc16, cols])
                            uv = plsc.load_gather(upd_v, [slot16, cols])
                            cd = plsc.load_gather(cur_v, [rr16, cols])
                            v = ev + rv + uv
                            cv = jnp.where(v >= 0.0, v, 0.01 * v)
                            acc = acc + cv * cd
                        return acc

                    acc = lax.fori_loop(0, 8, d_body, jnp.zeros((16,), jnp.float32))
                    sc_v[pl.ds(e0, 16)] = acc
                    return 0

                lax.fori_loop(0, CH // 16, group_body, 0)
            pltpu.sync_copy(sc_v, out_hbm.at[b])
            return 0

        lax.fori_loop(0, bpw, batch_body, 0)

    return score_k(ent_proj, rel_proj, upd33, slot_ids, cent_ids, crel_ids,
                   cur_state)


def _tc1_body(rows_aim_ref, rows_cur_ref, rel_ref, q_ref,
              wents_ref, wrels_ref, biass_ref, wn2_ref, wn3_ref, bn_ref,
              wc3_ref, bc_ref,
              gent_ref, greltab_ref, relproj_ref, curpart_ref):
    gent_ref[...] = rows_aim_ref[...] @ wents_ref[...] + biass_ref[...]
    greltab_ref[...] = rel_ref[...] @ wrels_ref[...]
    relproj_ref[...] = rel_ref[...] @ wc3_ref[...] + bc_ref[...]
    qp = q_ref[...] @ wn2_ref[...] + bn_ref[...]
    B = qp.shape[0]
    cp = rows_cur_ref[...] @ wn3_ref[...]
    curpart_ref[...] = cp.reshape(B, 32, 64) + qp[:, None, :]


def _tc2_body(et_ref, wc2_ref, out_ref):
    out_ref[...] = et_ref[...] @ wc2_ref[...]


def _tc3_body(upd_ref, curn_ref, aim_ref, curpart_ref, wc1_ref, wn1_ref,
              nid_ref, upd33_ref, cur_ref, slot_ref):
    B = aim_ref.shape[0]
    upd = upd_ref[...].reshape(B * 32, 64)
    upc = (upd @ wc1_ref[...]).reshape(B, 32, 64)
    upn = (upd @ wn1_ref[...]).reshape(B, 32, 64)
    upd33_ref[...] = jnp.concatenate(
        [upc, jnp.zeros((B, 1, 64), jnp.float32)], axis=1)
    eqn = nid_ref[...][:, :, None] == aim_ref[...][:, None, :]
    tp1 = lax.broadcasted_iota(jnp.int32, eqn.shape, 2) + 1
    mx = jnp.max(jnp.where(eqn, tp1, 0), axis=-1)
    slot_ref[...] = jnp.where(mx == 0, 32, mx - 1)
    eq = (curn_ref[...][:, :, None] == aim_ref[...][:, None, :]).astype(jnp.float32)
    tt = lax.broadcasted_iota(jnp.int32, (32, 32), 0)
    uu = (tt > lax.broadcasted_iota(jnp.int32, (32, 32), 1)).astype(jnp.float32)
    sfx = jax.lax.dot_general(eq, uu, (((2,), (0,)), ((), ())),
                              preferred_element_type=jnp.float32)
    onehot_last = eq * (sfx == 0.0).astype(jnp.float32)
    rep = jax.lax.dot_general(onehot_last, upn, (((2,), (1,)), ((0,), (0,))),
                              preferred_element_type=jnp.float32)
    pre = curpart_ref[...] + rep
    cur_ref[...] = jnp.where(pre >= 0.0, pre, 0.01 * pre) * 0.125


def _full_spec(shape):
    return pl.BlockSpec(shape, lambda: tuple(0 for _ in shape))


def kernel(start_entities, query_representations, aim_nodes, aim_entities,
           aim_nums, neighbors, neighbors_num, current_nodes, current_entities,
           current_masks, candidate_nodes, candidate_entities,
           candidate_relations, candidate_masks, entity_table, relation_table,
           W_ih, W_hh, b_ih, b_hh, W_next, b_next, W_cand, b_cand):
    B, TOPK = aim_nodes.shape
    ROLL = current_nodes.shape[1]
    MAXN = neighbors.shape[2]
    E = entity_table.shape[1]
    H = W_hh.shape[1]
    V = entity_table.shape[0]
    NREL = relation_table.shape[0]
    MAX_NODES = 2048
    f32 = jnp.float32

    i32 = jnp.int32
    idx_all = jnp.concatenate(
        [aim_entities.reshape(-1), current_entities.reshape(-1)]).astype(i32)
    rel_ids = neighbors[..., 1].reshape(B * TOPK, MAXN).astype(i32)
    nnum = neighbors_num.reshape(-1).astype(i32)
    cent_ids = candidate_entities.reshape(B, ROLL * MAXN).astype(i32)
    crel_ids = candidate_relations.reshape(B, ROLL * MAXN).astype(i32)
    cnode_ids = candidate_nodes.reshape(B, ROLL * MAXN).astype(i32)
    aim_i = aim_nodes.astype(i32)
    curn_i = current_nodes.astype(i32)

    scale = jnp.concatenate([-jnp.ones((H,), f32), jnp.ones((H,), f32),
                             2.0 * jnp.ones((H,), f32)])
    bias = b_ih + jnp.concatenate([b_hh[:2 * H], jnp.zeros((H,), f32)])
    W_ent_s = (W_ih[:, :E] * scale[:, None]).T
    W_rel_s = (W_ih[:, E:] * scale[:, None]).T
    bias_s = (bias * scale)[None, :]
    bhn2 = 2.0 * b_hh[2 * H:]
    Wn1T, Wn2T, Wn3T = (W_next[:, :H].T, W_next[:, H:H + 64].T,
                        W_next[:, H + 64:].T)
    Wc1T, Wc2T, Wc3T = (W_cand[:, :H].T, W_cand[:, H:H + E].T,
                        W_cand[:, H + E:].T)

    rows = _sc_gather_rows(entity_table, idx_all, idx_all.shape[0] // _NW, E)
    rows_aim = rows[:B * TOPK]
    rows_cur = rows[B * TOPK:]

    g_ent, g_rel_tab, rel_proj, cur_part = pl.pallas_call(
        _tc1_body,
        in_specs=[_full_spec((B * TOPK, E)), _full_spec((B * ROLL, E)),
                  _full_spec((NREL, E)), _full_spec((B, 64)),
                  _full_spec((E, 192)), _full_spec((E, 192)),
                  _full_spec((1, 192)), _full_spec((64, 64)),
                  _full_spec((64, 64)), _full_spec((1, 64)),
                  _full_spec((64, 64)), _full_spec((1, 64))],
        out_specs=[_full_spec((B * TOPK, 192)), _full_spec((NREL, 192)),
                   _full_spec((NREL, 64)), _full_spec((B, ROLL, 64))],
        out_shape=[jax.ShapeDtypeStruct((B * TOPK, 192), f32),
                   jax.ShapeDtypeStruct((NREL, 192), f32),
                   jax.ShapeDtypeStruct((NREL, 64), f32),
                   jax.ShapeDtypeStruct((B, ROLL, 64), f32)],
    )(rows_aim, rows_cur, relation_table, query_representations,
      W_ent_s, W_rel_s, bias_s, Wn2T, Wn3T, b_next[None, :],
      Wc3T, b_cand[None, :])

    RB = 800
    ent_proj = pl.pallas_call(
        _tc2_body,
        grid=(V // RB,),
        in_specs=[pl.BlockSpec((RB, E), lambda i: (i, 0)),
                  pl.BlockSpec((E, 64), lambda i: (0, 0))],
        out_specs=pl.BlockSpec((RB, 64), lambda i: (i, 0)),
        out_shape=jax.ShapeDtypeStruct((V, 64), f32),
    )(entity_table, Wc2T)

    upd = _sc_gru_mean(g_rel_tab, g_ent, rel_ids, nnum, bhn2)

    SB = 8
    upd33, cur_state, slot_ids = pl.pallas_call(
        _tc3_body,
        grid=(B // SB,),
        in_specs=[pl.BlockSpec((SB, TOPK, 64), lambda i: (i, 0, 0)),
                  pl.BlockSpec((SB, ROLL), lambda i: (i, 0)),
                  pl.BlockSpec((SB, TOPK), lambda i: (i, 0)),
                  pl.BlockSpec((SB, ROLL, 64), lambda i: (i, 0, 0)),
                  pl.BlockSpec((64, 64), lambda i: (0, 0)),
                  pl.BlockSpec((64, 64), lambda i: (0, 0)),
                  pl.BlockSpec((SB, ROLL * MAXN), lambda i: (i, 0))],
        out_specs=[pl.BlockSpec((SB, TOPK + 1, 64), lambda i: (i, 0, 0)),
                   pl.BlockSpec((SB, ROLL, 64), lambda i: (i, 0, 0)),
                   pl.BlockSpec((SB, ROLL * MAXN), lambda i: (i, 0))],
        out_shape=[jax.ShapeDtypeStruct((B, TOPK + 1, 64), f32),
                   jax.ShapeDtypeStruct((B, ROLL, 64), f32),
                   jax.ShapeDtypeStruct((B, ROLL * MAXN), i32)],
    )(upd.reshape(B, TOPK, 64), curn_i, aim_i, cur_part, Wc1T, Wn1T,
      cnode_ids)

    scores = _sc_score(ent_proj, rel_proj, upd33.reshape(B * (TOPK + 1), 64),
                       slot_ids, cent_ids, crel_ids,
                       cur_state.reshape(B * ROLL, 64))

    scores = scores.reshape(B, ROLL, MAXN)
    scores = jnp.where(~candidate_masks, -100000.0, scores)
    return scores


# speedup vs baseline: 1.0599x; 1.0549x over previous
import functools

import jax
import jax.numpy as jnp
from jax import lax
from jax.experimental import pallas as pl
from jax.experimental.pallas import tpu as pltpu
from jax.experimental.pallas import tpu_sc as plsc

_NC = 2
_NS = 16
_NW = _NC * _NS
_SC_PARAMS = pltpu.CompilerParams(use_tc_tiling_on_sc=False, needs_layout_passes=False)


def _mesh():
    return plsc.VectorSubcoreMesh(core_axis_name="c", subcore_axis_name="s")


def _rcp(x):
    i = plsc.bitcast(x, jnp.int32)
    y = plsc.bitcast(jnp.full(i.shape, 0x7EF311C3, jnp.int32) - i, jnp.float32)
    y = y * (2.0 - x * y)
    y = y * (2.0 - x * y)
    return y


def _rcp1(x):
    i = plsc.bitcast(x, jnp.int32)
    y = plsc.bitcast(jnp.full(i.shape, 0x7EF311C3, jnp.int32) - i, jnp.float32)
    return y * (2.0 - x * y)


def _wid():
    return lax.axis_index("s") * _NC + lax.axis_index("c")


def _sc_gather_rows(table, idx, rows_per_worker, dim):
    n = idx.shape[0]

    @functools.partial(
        pl.kernel,
        out_type=jax.ShapeDtypeStruct((n, dim), jnp.float32),
        mesh=_mesh(),
        scratch_types=[
            pltpu.VMEM((rows_per_worker,), jnp.int32),
            pltpu.VMEM((rows_per_worker, dim), jnp.float32),
            pltpu.SemaphoreType.DMA,
        ],
        compiler_params=_SC_PARAMS,
    )
    def gk(table_hbm, idx_hbm, out_hbm, idx_v, rows_v, sem):
        base = _wid() * rows_per_worker
        pltpu.sync_copy(idx_hbm.at[pl.ds(base, rows_per_worker)], idx_v)
        pltpu.async_copy(table_hbm.at[idx_v], rows_v, sem).wait()
        pltpu.sync_copy(rows_v, out_hbm.at[pl.ds(base, rows_per_worker)])

    return gk(table, idx)


def _sc_gru_mean(g_rel_tab, g_ent, rel_ids, nnum, bhn2):
    npairs, maxn = rel_ids.shape
    ppw = npairs // _NW

    @functools.partial(
        pl.kernel,
        out_type=jax.ShapeDtypeStruct((npairs, 64), jnp.float32),
        mesh=_mesh(),
        scratch_types=[
            pltpu.VMEM((ppw, maxn), jnp.int32),
            pltpu.VMEM((ppw, 192), jnp.float32),
            pltpu.VMEM((ppw,), jnp.int32),
            pltpu.VMEM((maxn, 192), jnp.float32),
            pltpu.VMEM((maxn, 192), jnp.float32),
            pltpu.VMEM((ppw, 64), jnp.float32),
            pltpu.VMEM((64,), jnp.float32),
            pltpu.SemaphoreType.DMA,
            pltpu.SemaphoreType.DMA,
        ],
        compiler_params=_SC_PARAMS,
    )
    def gru_k(greltab_hbm, gent_hbm, relids_hbm, nnum_hbm, bhn2_hbm, out_hbm,
              ids_v, gent_v, nn_v, grel0_v, grel1_v, upd_v, bhn_v, sem0, sem1):
        base = _wid() * ppw
        pltpu.sync_copy(relids_hbm.at[pl.ds(base, ppw)], ids_v)
        pltpu.sync_copy(gent_hbm.at[pl.ds(base, ppw)], gent_v)
        pltpu.sync_copy(nnum_hbm.at[pl.ds(base, ppw)], nn_v)
        pltpu.sync_copy(bhn2_hbm, bhn_v)
        bh = [bhn_v[pl.ds(j * 16, 16)] for j in range(4)]
        bufs = (grel0_v, grel1_v)
        sems = (sem0, sem1)

        def nn_of(p):
            return plsc.load_gather(nn_v, [jnp.full((16,), p, jnp.int32)])[0]

        def issue_pair(p, half):
            nnp = nn_of(p)
            for s in range(4):
                @pl.when(nnp > s * 16)
                def _():
                    pltpu.async_copy(
                        greltab_hbm.at[ids_v.at[p, pl.ds(s * 16, 16)]],
                        bufs[half].at[pl.ds(s * 16, 16)], sems[half])

        def drain_pair(p, half):
            nnp = nn_of(p)
            for s in range(4):
                @pl.when(nnp > s * 16)
                def _():
                    pltpu.make_async_copy(
                        greltab_hbm.at[pl.ds(0, 16)],
                        bufs[half].at[pl.ds(s * 16, 16)], sems[half]).wait()

        issue_pair(0, 0)

        def p2_body(p2, _):
            for half in range(2):
                p = p2 * 2 + half
                grel_v = bufs[half]
                drain_pair(p, half)

                @pl.when(p + 1 < ppw)
                def _():
                    issue_pair(p + 1, 1 - half)

                nn16 = plsc.load_gather(nn_v, [jnp.full((16,), p, jnp.int32)])
                nn = nn16[0]
                ge = [gent_v[p, pl.ds(j * 16, 16)] for j in range(12)]

                def gate(m, j):
                    pr = grel_v[m, pl.ds(j * 16, 16)] + ge[j]
                    pz = grel_v[m, pl.ds(64 + j * 16, 16)] + ge[4 + j]
                    pn = grel_v[m, pl.ds(128 + j * 16, 16)] + ge[8 + j]
                    r = _rcp1(1.0 + jnp.exp(pr))
                    e2 = jnp.exp(jnp.minimum(pn + r * bh[j], 60.0))
                    return (e2 - 1.0) * _rcp1((e2 + 1.0) * (1.0 + jnp.exp(pz)))

                def m2_body(i, accs):
                    m = i * 2
                    return tuple(accs[j] + gate(m, j) + gate(m + 1, j)
                                 for j in range(4))

                z16 = jnp.zeros((16,), jnp.float32)
                accs = lax.fori_loop(
                    0, lax.shift_right_logical(nn, 1), m2_body,
                    (z16, z16, z16, z16))
                modd = jnp.maximum(nn - 1, 0)
                oddf = (lax.bitwise_and(nn, 1)).astype(jnp.float32)
                accs = tuple(accs[j] + oddf * gate(modd, j) for j in range(4))
                inv16 = _rcp(jnp.maximum(nn16.astype(jnp.float32), 1.0))
                for j in range(4):
                    upd_v[p, pl.ds(j * 16, 16)] = accs[j] * inv16
            return 0

        lax.fori_loop(0, ppw // 2, p2_body, 0)
        pltpu.sync_copy(upd_v, out_hbm.at[pl.ds(base, ppw)])

    return gru_k(g_rel_tab, g_ent, rel_ids, nnum, bhn2)


def _sc_score(ent_proj, rel_proj, upd33, slot_ids, cent_ids, crel_ids,
              cur_state):
    B, NE = cent_ids.shape
    bpw = B // _NW
    CH = 256
    nch = NE // CH

    @functools.partial(
        pl.kernel,
        out_type=jax.ShapeDtypeStruct((B, NE), jnp.float32),
        mesh=_mesh(),
        scratch_types=[
            pltpu.VMEM((33, 64), jnp.float32),
            pltpu.VMEM((32, 64), jnp.float32),
            pltpu.VMEM((NE,), jnp.int32),
            pltpu.VMEM((NE,), jnp.int32),
            pltpu.VMEM((NE,), jnp.int32),
            pltpu.VMEM((CH, 64), jnp.float32),
            pltpu.VMEM((CH, 64), jnp.float32),
            pltpu.VMEM((CH, 64), jnp.float32),
            pltpu.VMEM((CH, 64), jnp.float32),
            pltpu.VMEM((NE,), jnp.float32),
            pltpu.SemaphoreType.DMA,
            pltpu.SemaphoreType.DMA,
            pltpu.SemaphoreType.DMA,
            pltpu.SemaphoreType.DMA,
        ],
        compiler_params=_SC_PARAMS,
    )
    def score_k(entp_hbm, relp_hbm, upd_hbm, slot_hbm, cent_hbm, crel_hbm,
                cur_hbm, out_hbm, upd_v, cur_v,
                sid_v, eid_v, rid_v, erow0_v, erow1_v, rrow0_v, rrow1_v,
                sc_v, sem_e0, sem_e1, sem_r0, sem_r1):
        wid = _wid()
        ebufs, rbufs = (erow0_v, erow1_v), (rrow0_v, rrow1_v)
        esems, rsems = (sem_e0, sem_e1), (sem_r0, sem_r1)

        def issue(c, bi):
            pltpu.async_copy(
                entp_hbm.at[eid_v.at[pl.ds(c * CH, CH)]], ebufs[bi], esems[bi])
            pltpu.async_copy(
                relp_hbm.at[rid_v.at[pl.ds(c * CH, CH)]], rbufs[bi], rsems[bi])

        def batch_body(k, _):
            b = wid * bpw + k
            pltpu.sync_copy(upd_hbm.at[pl.ds(b * 33, 33)], upd_v)
            pltpu.sync_copy(cur_hbm.at[pl.ds(b * 32, 32)], cur_v)
            pltpu.sync_copy(slot_hbm.at[b], sid_v)
            pltpu.sync_copy(cent_hbm.at[b], eid_v)
            pltpu.sync_copy(crel_hbm.at[b], rid_v)

            lane = lax.iota(jnp.int32, 16)
            issue(0, 0)
            for c in range(nch):
                bi = c & 1
                erow_v, rrow_v = ebufs[bi], rbufs[bi]
                pltpu.make_async_copy(
                    entp_hbm.at[pl.ds(0, CH)], erow_v, esems[bi]).wait()
                pltpu.make_async_copy(
                    relp_hbm.at[pl.ds(0, CH)], rrow_v, rsems[bi]).wait()
                if c + 1 < nch:
                    issue(c + 1, 1 - bi)

                def group_body(gg, _, c=c, erow_v=erow_v, rrow_v=rrow_v):
                    e0 = c * CH + gg * 16
                    slot16 = sid_v[pl.ds(e0, 16)]
                    loc16 = lane + gg * 16
                    r_row = lax.shift_right_logical(c * CH + gg * 16, 6)
                    rr16 = jnp.full((16,), r_row, jnp.int32)

                    def d_body(i, acc):
                        for dd in range(8):
                            cols = lax.bitwise_and(lane + (i * 8 + dd), 63)
                            ev = plsc.load_gather(erow_v, [loc16, cols])
                            rv = plsc.load_gather(rrow_v, [loc16, cols])
                            uv = plsc.load_gather(upd_v, [slot16, cols])
                            cd = plsc.load_gather(cur_v, [rr16, cols])
                            v = ev + rv + uv
                            cv = jnp.where(v >= 0.0, v, 0.01 * v)
                            acc = acc + cv * cd
                        return acc

                    acc = lax.fori_loop(0, 8, d_body, jnp.zeros((16,), jnp.float32))
                    sc_v[pl.ds(e0, 16)] = acc
                    return 0

                lax.fori_loop(0, CH // 16, group_body, 0)
            pltpu.sync_copy(sc_v, out_hbm.at[b])
            return 0

        lax.fori_loop(0, bpw, batch_body, 0)

    return score_k(ent_proj, rel_proj, upd33, slot_ids, cent_ids, crel_ids,
                   cur_state)


def _tc1_body(rows_aim_ref, rows_cur_ref, rel_ref, q_ref,
              wents_ref, wrels_ref, biass_ref, wn2_ref, wn3_ref, bn_ref,
              wc3_ref, bc_ref,
              gent_ref, greltab_ref, relproj_ref, curpart_ref):
    gent_ref[...] = rows_aim_ref[...] @ wents_ref[...] + biass_ref[...]
    greltab_ref[...] = rel_ref[...] @ wrels_ref[...]
    relproj_ref[...] = rel_ref[...] @ wc3_ref[...] + bc_ref[...]
    qp = q_ref[...] @ wn2_ref[...] + bn_ref[...]
    B = qp.shape[0]
    cp = rows_cur_ref[...] @ wn3_ref[...]
    curpart_ref[...] = cp.reshape(B, 32, 64) + qp[:, None, :]


def _tc2_body(et_ref, wc2_ref, out_ref):
    out_ref[...] = et_ref[...] @ wc2_ref[...]


def _tc3_body(upd_ref, curn_ref, aim_ref, curpart_ref, wc1_ref, wn1_ref,
              nid_ref, upd33_ref, cur_ref, slot_ref):
    B = aim_ref.shape[0]
    upd = upd_ref[...].reshape(B * 32, 64)
    upc = (upd @ wc1_ref[...]).reshape(B, 32, 64)
    upn = (upd @ wn1_ref[...]).reshape(B, 32, 64)
    upd33_ref[...] = jnp.concatenate(
        [upc, jnp.zeros((B, 1, 64), jnp.float32)], axis=1)
    eqn = nid_ref[...][:, :, None] == aim_ref[...][:, None, :]
    tp1 = lax.broadcasted_iota(jnp.int32, eqn.shape, 2) + 1
    mx = jnp.max(jnp.where(eqn, tp1, 0), axis=-1)
    slot_ref[...] = jnp.where(mx == 0, 32, mx - 1)
    eq = (curn_ref[...][:, :, None] == aim_ref[...][:, None, :]).astype(jnp.float32)
    tt = lax.broadcasted_iota(jnp.int32, (32, 32), 0)
    uu = (tt > lax.broadcasted_iota(jnp.int32, (32, 32), 1)).astype(jnp.float32)
    sfx = jax.lax.dot_general(eq, uu, (((2,), (0,)), ((), ())),
                              preferred_element_type=jnp.float32)
    onehot_last = eq * (sfx == 0.0).astype(jnp.float32)
    rep = jax.lax.dot_general(onehot_last, upn, (((2,), (1,)), ((0,), (0,))),
                              preferred_element_type=jnp.float32)
    pre = curpart_ref[...] + rep
    cur_ref[...] = jnp.where(pre >= 0.0, pre, 0.01 * pre) * 0.125


def _full_spec(shape):
    return pl.BlockSpec(shape, lambda: tuple(0 for _ in shape))


def kernel(start_entities, query_representations, aim_nodes, aim_entities,
           aim_nums, neighbors, neighbors_num, current_nodes, current_entities,
           current_masks, candidate_nodes, candidate_entities,
           candidate_relations, candidate_masks, entity_table, relation_table,
           W_ih, W_hh, b_ih, b_hh, W_next, b_next, W_cand, b_cand):
    B, TOPK = aim_nodes.shape
    ROLL = current_nodes.shape[1]
    MAXN = neighbors.shape[2]
    E = entity_table.shape[1]
    H = W_hh.shape[1]
    V = entity_table.shape[0]
    NREL = relation_table.shape[0]
    MAX_NODES = 2048
    f32 = jnp.float32

    i32 = jnp.int32
    idx_all = jnp.concatenate(
        [aim_entities.reshape(-1), current_entities.reshape(-1)]).astype(i32)
    rel_ids = neighbors[..., 1].reshape(B * TOPK, MAXN).astype(i32)
    nnum = neighbors_num.reshape(-1).astype(i32)
    cent_ids = candidate_entities.reshape(B, ROLL * MAXN).astype(i32)
    crel_ids = candidate_relations.reshape(B, ROLL * MAXN).astype(i32)
    cnode_ids = candidate_nodes.reshape(B, ROLL * MAXN).astype(i32)
    aim_i = aim_nodes.astype(i32)
    curn_i = current_nodes.astype(i32)

    scale = jnp.concatenate([-jnp.ones((H,), f32), jnp.ones((H,), f32),
                             2.0 * jnp.ones((H,), f32)])
    bias = b_ih + jnp.concatenate([b_hh[:2 * H], jnp.zeros((H,), f32)])
    W_ent_s = (W_ih[:, :E] * scale[:, None]).T
    W_rel_s = (W_ih[:, E:] * scale[:, None]).T
    bias_s = (bias * scale)[None, :]
    bhn2 = 2.0 * b_hh[2 * H:]
    Wn1T, Wn2T, Wn3T = (W_next[:, :H].T, W_next[:, H:H + 64].T,
                        W_next[:, H + 64:].T)
    Wc1T, Wc2T, Wc3T = (W_cand[:, :H].T, W_cand[:, H:H + E].T,
                        W_cand[:, H + E:].T)

    rows = _sc_gather_rows(entity_table, idx_all, idx_all.shape[0] // _NW, E)
    rows_aim = rows[:B * TOPK]
    rows_cur = rows[B * TOPK:]

    g_ent, g_rel_tab, rel_proj, cur_part = pl.pallas_call(
        _tc1_body,
        in_specs=[_full_spec((B * TOPK, E)), _full_spec((B * ROLL, E)),
                  _full_spec((NREL, E)), _full_spec((B, 64)),
                  _full_spec((E, 192)), _full_spec((E, 192)),
                  _full_spec((1, 192)), _full_spec((64, 64)),
                  _full_spec((64, 64)), _full_spec((1, 64)),
                  _full_spec((64, 64)), _full_spec((1, 64))],
        out_specs=[_full_spec((B * TOPK, 192)), _full_spec((NREL, 192)),
                   _full_spec((NREL, 64)), _full_spec((B, ROLL, 64))],
        out_shape=[jax.ShapeDtypeStruct((B * TOPK, 192), f32),
                   jax.ShapeDtypeStruct((NREL, 192), f32),
                   jax.ShapeDtypeStruct((NREL, 64), f32),
                   jax.ShapeDtypeStruct((B, ROLL, 64), f32)],
    )(rows_aim, rows_cur, relation_table, query_representations,
      W_ent_s, W_rel_s, bias_s, Wn2T, Wn3T, b_next[None, :],
      Wc3T, b_cand[None, :])

    RB = 800
    ent_proj = pl.pallas_call(
        _tc2_body,
        grid=(V // RB,),
        in_specs=[pl.BlockSpec((RB, E), lambda i: (i, 0)),
                  pl.BlockSpec((E, 64), lambda i: (0, 0))],
        out_specs=pl.BlockSpec((RB, 64), lambda i: (i, 0)),
        out_shape=jax.ShapeDtypeStruct((V, 64), f32),
    )(entity_table, Wc2T)

    upd = _sc_gru_mean(g_rel_tab, g_ent, rel_ids, nnum, bhn2)

    SB = 8
    upd33, cur_state, slot_ids = pl.pallas_call(
        _tc3_body,
        grid=(B // SB,),
        in_specs=[pl.BlockSpec((SB, TOPK, 64), lambda i: (i, 0, 0)),
                  pl.BlockSpec((SB, ROLL), lambda i: (i, 0)),
                  pl.BlockSpec((SB, TOPK), lambda i: (i, 0)),
                  pl.BlockSpec((SB, ROLL, 64), lambda i: (i, 0, 0)),
                  pl.BlockSpec((64, 64), lambda i: (0, 0)),
                  pl.BlockSpec((64, 64), lambda i: (0, 0)),
                  pl.BlockSpec((SB, ROLL * MAXN), lambda i: (i, 0))],
        out_specs=[pl.BlockSpec((SB, TOPK + 1, 64), lambda i: (i, 0, 0)),
                   pl.BlockSpec((SB, ROLL, 64), lambda i: (i, 0, 0)),
                   pl.BlockSpec((SB, ROLL * MAXN), lambda i: (i, 0))],
        out_shape=[jax.ShapeDtypeStruct((B, TOPK + 1, 64), f32),
                   jax.ShapeDtypeStruct((B, ROLL, 64), f32),
                   jax.ShapeDtypeStruct((B, ROLL * MAXN), i32)],
    )(upd.reshape(B, TOPK, 64), curn_i, aim_i, cur_part, Wc1T, Wn1T,
      cnode_ids)

    scores = _sc_score(ent_proj, rel_proj, upd33.reshape(B * (TOPK + 1), 64),
                       slot_ids, cent_ids, crel_ids,
                       cur_state.reshape(B * ROLL, 64))

    scores = scores.reshape(B, ROLL, MAXN)
    scores = jnp.where(~candidate_masks, -100000.0, scores)
    return scores
